# baseline (speedup 1.0000x reference)
import functools

import jax
import jax.numpy as jnp
from jax import lax
from jax.experimental import pallas as pl
from jax.experimental.pallas import tpu as pltpu
from jax.experimental.pallas import tpu_sc as plsc

_N, _D, _H, _C, _E = 10000, 128, 16, 40, 320000
_NC, _NS = 2, 16
_NW = _NC * _NS
_EB = 128
_CH = 80
_G = 4
_EPAD = _NW * _CH * _EB
_ACC = 10240
_ZR = _ACC // _NS
_OR = _N // _NS


def _seg_body(table, srcr, dstr, out,
              src_idx, dst_idx, rows0, rows1, stage, acc, gsem, ssem):
    cid = lax.axis_index("c")
    sid = lax.axis_index("s")
    wid = cid * _NS + sid

    def _zero(i, c):
        stage[i] = jnp.zeros((_H,), jnp.float32)
        return c
    lax.fori_loop(0, _ZR, _zero, 0)
    pltpu.sync_copy(stage, acc.at[pl.ds(sid * _ZR, _ZR)])
    plsc.subcore_barrier()

    base = wid * _CH
    pltpu.sync_copy(srcr.at[pl.ds(base, _CH)], src_idx)
    pltpu.sync_copy(dstr.at[pl.ds(base, _CH)], dst_idx)

    rows = (rows0, rows1)

    def _gather(g, j, b):
        pltpu.async_copy(table.at[src_idx.at[j]], rows[g].at[b], gsem)

    def _gwait(g, j, b):
        pltpu.make_async_copy(table.at[src_idx.at[j]], rows[g].at[b],
                              gsem).wait()

    def _scatter(g, j, b):
        pltpu.async_copy(rows[g].at[b], acc.at[dst_idx.at[j]], ssem, add=True)

    def _swait(g, j, b):
        pltpu.make_async_copy(rows[g].at[b], acc.at[dst_idx.at[j]],
                              ssem).wait()

    for b in range(_G):
        _gather(0, b, b)

    def _phase(g, base):
        for b in range(_G):
            _gwait(g, base + b, b)
        for b in range(_G):
            _scatter(g, base + b, b)

        @pl.when(base > 0)
        def _():
            for b in range(_G):
                _swait(1 - g, base - _G + b, b)

        @pl.when(base + _G < _CH)
        def _():
            for b in range(_G):
                _gather(1 - g, base + _G + b, b)

    def _pair(i, c):
        base = i * (2 * _G)
        _phase(0, base)
        _phase(1, base + _G)
        return c

    lax.fori_loop(0, _CH // (2 * _G), _pair, 0)
    for b in range(_G):
        _swait((_CH // _G - 1) % 2, _CH - _G + b, b)
    plsc.subcore_barrier()

    pltpu.sync_copy(acc.at[pl.ds(sid * _ZR, _ZR)],
                    out.at[cid, pl.ds(sid * _ZR, _ZR)])


_segsum = functools.partial(
    pl.kernel,
    out_type=jax.ShapeDtypeStruct((_NC, _ACC, _H), jnp.float32),
    mesh=plsc.VectorSubcoreMesh(core_axis_name="c", subcore_axis_name="s"),
    scratch_types=[
        pltpu.VMEM((_CH, _EB), jnp.int32),
        pltpu.VMEM((_CH, _EB), jnp.int32),
        pltpu.VMEM((_G, _EB, _H), jnp.float32),
        pltpu.VMEM((_G, _EB, _H), jnp.float32),
        pltpu.VMEM((_ZR, _H), jnp.float32),
        pltpu.VMEM_SHARED((_ACC, _H), jnp.float32),
        pltpu.SemaphoreType.DMA,
        pltpu.SemaphoreType.DMA,
    ],
    compiler_params=pltpu.CompilerParams(use_tc_tiling_on_sc=False),
)(_seg_body)


def _lin1_body(x_ref, w_ref, o_ref):
    o_ref[...] = jnp.dot(x_ref[...], w_ref[...],
                         preferred_element_type=jnp.float32,
                         precision=lax.Precision.HIGHEST)


def _relu_add_body(y_ref, p_ref, b_ref, o_ref):
    s = y_ref[...] + p_ref[0, :_N] + p_ref[1, :_N] + b_ref[...]
    o_ref[...] = jnp.maximum(s, 0.0)


def _out_body(h_ref, q_ref, w_ref, b_ref, o_ref):
    t = h_ref[...] + q_ref[0, :_N] + q_ref[1, :_N]
    z = jnp.dot(t, w_ref[...], preferred_element_type=jnp.float32,
                precision=lax.Precision.HIGHEST) + b_ref[...]
    z = jnp.maximum(z, 0.0)
    m = jnp.max(z, axis=1, keepdims=True)
    z = z - m
    o_ref[...] = z - jnp.log(jnp.sum(jnp.exp(z), axis=1, keepdims=True))


def kernel(x, edge_index, W1, b1, W2, b2):
    src = edge_index[0]
    dst = edge_index[1]
    pad = _EPAD - _E
    srcr = jnp.concatenate(
        [src, jnp.zeros((pad,), jnp.int32)]).reshape(_NW * _CH, _EB)
    dstr = jnp.concatenate(
        [dst, jnp.full((pad,), _N, jnp.int32)]).reshape(_NW * _CH, _EB)

    y1 = pl.pallas_call(
        _lin1_body,
        out_shape=jax.ShapeDtypeStruct((_N, _H), jnp.float32),
    )(x, W1)

    p = _segsum(y1, srcr, dstr)

    h1 = pl.pallas_call(
        _relu_add_body,
        out_shape=jax.ShapeDtypeStruct((_N, _H), jnp.float32),
    )(y1, p, b1.reshape(1, _H))

    q = _segsum(h1, srcr, dstr)

    out = pl.pallas_call(
        _out_body,
        out_shape=jax.ShapeDtypeStruct((_N, _C), jnp.float32),
    )(h1, q, W2, b2.reshape(1, _C))
    return out


# speedup vs baseline: 1.2218x; 1.2218x over previous
import functools

import jax
import jax.numpy as jnp
from jax import lax
from jax.experimental import pallas as pl
from jax.experimental.pallas import tpu as pltpu
from jax.experimental.pallas import tpu_sc as plsc

_N, _D, _H, _C, _E = 10000, 128, 16, 40, 320000
_NC, _NS = 2, 16
_EB = 128
_ER = _E // _EB
_B0 = 90
_B1 = 66
_NX = _ER - _NS * (_B0 + _B1)
_ACC = 10240
_ZR = _ACC // _NS
_BMAX = _B0 + 1


def _seg_body(table, er, out, src_idx, dst_idx, rows0, rows1, stage, acc,
              sem0, sem1):
    cid = lax.axis_index("c")
    sid = lax.axis_index("s")

    def _zero(i, c):
        stage[i] = jnp.zeros((_H,), jnp.float32)
        return c
    lax.fori_loop(0, _ZR, _zero, 0)
    pltpu.sync_copy(stage, acc.at[pl.ds(sid * _ZR, _ZR)])
    plsc.subcore_barrier()

    def _edge_loop(base, nb):
        pltpu.sync_copy(er.at[0, pl.ds(base, nb)], src_idx.at[pl.ds(0, nb)])
        pltpu.sync_copy(er.at[1, pl.ds(base, nb)], dst_idx.at[pl.ds(0, nb)])

        pltpu.async_copy(table.at[src_idx.at[0]], rows0, sem0)

        def _step(i, c):
            j = 2 * i
            pltpu.async_copy(table.at[src_idx.at[j + 1]], rows1, sem1)
            pltpu.make_async_copy(table.at[src_idx.at[j]], rows0, sem0).wait()
            pltpu.sync_copy(rows0, acc.at[dst_idx.at[j]], add=True)

            @pl.when(j + 2 < nb)
            def _():
                pltpu.async_copy(table.at[src_idx.at[j + 2]], rows0, sem0)

            pltpu.make_async_copy(table.at[src_idx.at[j + 1]], rows1,
                                  sem1).wait()
            pltpu.sync_copy(rows1, acc.at[dst_idx.at[j + 1]], add=True)
            return c

        lax.fori_loop(0, nb // 2, _step, 0)

    @pl.when(cid == 0)
    def _():
        _edge_loop(sid * _B0, _B0)

        @pl.when(sid < _NX)
        def _():
            xb = _NS * _B0 + _NS * _B1 + sid
            pltpu.sync_copy(er.at[0, pl.ds(xb, 1)], src_idx.at[pl.ds(0, 1)])
            pltpu.sync_copy(er.at[1, pl.ds(xb, 1)], dst_idx.at[pl.ds(0, 1)])
            pltpu.async_copy(table.at[src_idx.at[0]], rows0, sem0).wait()
            pltpu.sync_copy(rows0, acc.at[dst_idx.at[0]], add=True)

    @pl.when(cid == 1)
    def _():
        _edge_loop(_NS * _B0 + sid * _B1, _B1)

    plsc.subcore_barrier()

    pltpu.sync_copy(acc.at[pl.ds(sid * _ZR, _ZR)],
                    out.at[cid, pl.ds(sid * _ZR, _ZR)])


_segsum = functools.partial(
    pl.kernel,
    out_type=jax.ShapeDtypeStruct((_NC, _ACC, _H), jnp.float32),
    mesh=plsc.VectorSubcoreMesh(core_axis_name="c", subcore_axis_name="s"),
    scratch_types=[
        pltpu.VMEM((_BMAX, _EB), jnp.int32),
        pltpu.VMEM((_BMAX, _EB), jnp.int32),
        pltpu.VMEM((_EB, _H), jnp.float32),
        pltpu.VMEM((_EB, _H), jnp.float32),
        pltpu.VMEM((_ZR, _H), jnp.float32),
        pltpu.VMEM_SHARED((_ACC, _H), jnp.float32),
        pltpu.SemaphoreType.DMA,
        pltpu.SemaphoreType.DMA,
    ],
    compiler_params=pltpu.CompilerParams(use_tc_tiling_on_sc=False),
)(_seg_body)


def _lin1_body(x_ref, w_ref, o_ref):
    o_ref[...] = jnp.dot(x_ref[...], w_ref[...],
                         preferred_element_type=jnp.float32,
                         precision=lax.Precision.HIGHEST)


def _relu_add_body(y_ref, p_ref, b_ref, o_ref):
    s = y_ref[...] + p_ref[0, :_N] + p_ref[1, :_N] + b_ref[...]
    o_ref[...] = jnp.maximum(s, 0.0)


def _out_body(h_ref, q_ref, w_ref, b_ref, o_ref):
    t = h_ref[...] + q_ref[0, :_N] + q_ref[1, :_N]
    z = jnp.dot(t, w_ref[...], preferred_element_type=jnp.float32,
                precision=lax.Precision.HIGHEST) + b_ref[...]
    z = jnp.maximum(z, 0.0)
    m = jnp.max(z, axis=1, keepdims=True)
    z = z - m
    o_ref[...] = z - jnp.log(jnp.sum(jnp.exp(z), axis=1, keepdims=True))


def kernel(x, edge_index, W1, b1, W2, b2):
    er = edge_index.reshape(2, _ER, _EB)

    y1 = pl.pallas_call(
        _lin1_body,
        out_shape=jax.ShapeDtypeStruct((_N, _H), jnp.float32),
    )(x, W1)

    p = _segsum(y1, er)

    h1 = pl.pallas_call(
        _relu_add_body,
        out_shape=jax.ShapeDtypeStruct((_N, _H), jnp.float32),
    )(y1, p, b1.reshape(1, _H))

    q = _segsum(h1, er)

    out = pl.pallas_call(
        _out_body,
        out_shape=jax.ShapeDtypeStruct((_N, _C), jnp.float32),
    )(h1, q, W2, b2.reshape(1, _C))
    return out


# speedup vs baseline: 1.2836x; 1.0506x over previous
import functools

import jax
import jax.numpy as jnp
from jax import lax
from jax.experimental import pallas as pl
from jax.experimental.pallas import tpu as pltpu
from jax.experimental.pallas import tpu_sc as plsc

_N, _D, _H, _C, _E = 10000, 128, 16, 40, 320000
_NC, _NS = 2, 16
_EB = 128
_ER = _E // _EB
_B0 = 80
_B1 = 76
_NX = _ER - _NS * (_B0 + _B1)
_ACC = 10240
_ZR = _ACC // _NS
_BMAX = _B0 + 1


def _seg_body(table, er, out, src_idx, dst_idx, rows0, rows1, stage, acc,
              sem0, sem1):
    cid = lax.axis_index("c")
    sid = lax.axis_index("s")

    def _zero(i, c):
        stage[i] = jnp.zeros((_H,), jnp.float32)
        return c
    lax.fori_loop(0, _ZR, _zero, 0)
    pltpu.sync_copy(stage, acc.at[pl.ds(sid * _ZR, _ZR)])
    plsc.subcore_barrier()

    def _edge_loop(base, nb):
        pltpu.sync_copy(er.at[0, pl.ds(base, nb)], src_idx.at[pl.ds(0, nb)])
        pltpu.sync_copy(er.at[1, pl.ds(base, nb)], dst_idx.at[pl.ds(0, nb)])

        pltpu.async_copy(table.at[src_idx.at[0]], rows0, sem0)

        def _step(i, c):
            j = 2 * i
            pltpu.async_copy(table.at[src_idx.at[j + 1]], rows1, sem1)
            pltpu.make_async_copy(table.at[src_idx.at[j]], rows0, sem0).wait()
            pltpu.sync_copy(rows0, acc.at[dst_idx.at[j]], add=True)

            @pl.when(j + 2 < nb)
            def _():
                pltpu.async_copy(table.at[src_idx.at[j + 2]], rows0, sem0)

            pltpu.make_async_copy(table.at[src_idx.at[j + 1]], rows1,
                                  sem1).wait()
            pltpu.sync_copy(rows1, acc.at[dst_idx.at[j + 1]], add=True)
            return c

        lax.fori_loop(0, nb // 2, _step, 0)

    @pl.when(cid == 0)
    def _():
        _edge_loop(sid * _B0, _B0)

        @pl.when(sid < _NX)
        def _():
            xb = _NS * _B0 + _NS * _B1 + sid
            pltpu.sync_copy(er.at[0, pl.ds(xb, 1)], src_idx.at[pl.ds(0, 1)])
            pltpu.sync_copy(er.at[1, pl.ds(xb, 1)], dst_idx.at[pl.ds(0, 1)])
            pltpu.async_copy(table.at[src_idx.at[0]], rows0, sem0).wait()
            pltpu.sync_copy(rows0, acc.at[dst_idx.at[0]], add=True)

    @pl.when(cid == 1)
    def _():
        _edge_loop(_NS * _B0 + sid * _B1, _B1)

    plsc.subcore_barrier()

    pltpu.sync_copy(acc.at[pl.ds(sid * _ZR, _ZR)],
                    out.at[cid, pl.ds(sid * _ZR, _ZR)])


_segsum = functools.partial(
    pl.kernel,
    out_type=jax.ShapeDtypeStruct((_NC, _ACC, _H), jnp.float32),
    mesh=plsc.VectorSubcoreMesh(core_axis_name="c", subcore_axis_name="s"),
    scratch_types=[
        pltpu.VMEM((_BMAX, _EB), jnp.int32),
        pltpu.VMEM((_BMAX, _EB), jnp.int32),
        pltpu.VMEM((_EB, _H), jnp.float32),
        pltpu.VMEM((_EB, _H), jnp.float32),
        pltpu.VMEM((_ZR, _H), jnp.float32),
        pltpu.VMEM_SHARED((_ACC, _H), jnp.float32),
        pltpu.SemaphoreType.DMA,
        pltpu.SemaphoreType.DMA,
    ],
    compiler_params=pltpu.CompilerParams(use_tc_tiling_on_sc=False),
)(_seg_body)


def _lin1_body(x_ref, w_ref, o_ref):
    o_ref[...] = jnp.dot(x_ref[...], w_ref[...],
                         preferred_element_type=jnp.float32,
                         precision=lax.Precision.HIGHEST)


def _relu_add_body(y_ref, p_ref, b_ref, o_ref):
    s = y_ref[...] + p_ref[0, :_N] + p_ref[1, :_N] + b_ref[...]
    o_ref[...] = jnp.maximum(s, 0.0)


def _out_body(h_ref, q_ref, w_ref, b_ref, o_ref):
    t = h_ref[...] + q_ref[0, :_N] + q_ref[1, :_N]
    z = jnp.dot(t, w_ref[...], preferred_element_type=jnp.float32,
                precision=lax.Precision.HIGHEST) + b_ref[...]
    z = jnp.maximum(z, 0.0)
    m = jnp.max(z, axis=1, keepdims=True)
    z = z - m
    o_ref[...] = z - jnp.log(jnp.sum(jnp.exp(z), axis=1, keepdims=True))


def kernel(x, edge_index, W1, b1, W2, b2):
    er = edge_index.reshape(2, _ER, _EB)

    y1 = pl.pallas_call(
        _lin1_body,
        out_shape=jax.ShapeDtypeStruct((_N, _H), jnp.float32),
    )(x, W1)

    p = _segsum(y1, er)

    h1 = pl.pallas_call(
        _relu_add_body,
        out_shape=jax.ShapeDtypeStruct((_N, _H), jnp.float32),
    )(y1, p, b1.reshape(1, _H))

    q = _segsum(h1, er)

    out = pl.pallas_call(
        _out_body,
        out_shape=jax.ShapeDtypeStruct((_N, _C), jnp.float32),
    )(h1, q, W2, b2.reshape(1, _C))
    return out


# speedup vs baseline: 1.3395x; 1.0436x over previous
import functools

import jax
import jax.numpy as jnp
from jax import lax
from jax.experimental import pallas as pl
from jax.experimental.pallas import tpu as pltpu
from jax.experimental.pallas import tpu_sc as plsc

_N, _D, _H, _C, _E = 10000, 128, 16, 40, 320000
_NC, _NS = 2, 16
_EB = 128
_ER = _E // _EB
_B0 = 80
_B1 = 76
_NX = _ER - _NS * (_B0 + _B1)
_ACC = 10240
_ZR = _ACC // _NS
_BMAX = _B0 + 1


def _seg_body(table, er, out, src_idx, dst_idx, rows0, rows1, stage, acc,
              sem0, sem1):
    cid = lax.axis_index("c")
    sid = lax.axis_index("s")

    def _zero(i, c):
        stage[i] = jnp.zeros((_H,), jnp.float32)
        return c
    lax.fori_loop(0, _ZR, _zero, 0)
    pltpu.sync_copy(stage, acc.at[pl.ds(sid * _ZR, _ZR)])
    plsc.subcore_barrier()

    def _edge_loop(base, nb):
        pltpu.sync_copy(er.at[0, pl.ds(base, nb)], src_idx.at[pl.ds(0, nb)])
        pltpu.sync_copy(er.at[1, pl.ds(base, nb)], dst_idx.at[pl.ds(0, nb)])

        pltpu.async_copy(table.at[src_idx.at[0]], rows0, sem0)

        def _step(i, c):
            j = 2 * i
            pltpu.async_copy(table.at[src_idx.at[j + 1]], rows1, sem1)
            pltpu.make_async_copy(table.at[src_idx.at[j]], rows0, sem0).wait()
            pltpu.sync_copy(rows0, acc.at[dst_idx.at[j]], add=True)

            @pl.when(j + 2 < nb)
            def _():
                pltpu.async_copy(table.at[src_idx.at[j + 2]], rows0, sem0)

            pltpu.make_async_copy(table.at[src_idx.at[j + 1]], rows1,
                                  sem1).wait()
            pltpu.sync_copy(rows1, acc.at[dst_idx.at[j + 1]], add=True)
            return c

        lax.fori_loop(0, nb // 2, _step, 0)

    @pl.when(cid == 0)
    def _():
        _edge_loop(sid * _B0, _B0)

        @pl.when(sid < _NX)
        def _():
            xb = _NS * _B0 + _NS * _B1 + sid
            pltpu.sync_copy(er.at[0, pl.ds(xb, 1)], src_idx.at[pl.ds(0, 1)])
            pltpu.sync_copy(er.at[1, pl.ds(xb, 1)], dst_idx.at[pl.ds(0, 1)])
            pltpu.async_copy(table.at[src_idx.at[0]], rows0, sem0).wait()
            pltpu.sync_copy(rows0, acc.at[dst_idx.at[0]], add=True)

    @pl.when(cid == 1)
    def _():
        _edge_loop(_NS * _B0 + sid * _B1, _B1)

    plsc.subcore_barrier()

    pltpu.sync_copy(acc.at[pl.ds(sid * _ZR, _ZR)],
                    out.at[cid, pl.ds(sid * _ZR, _ZR)])


_segsum = functools.partial(
    pl.kernel,
    out_type=jax.ShapeDtypeStruct((_NC, _ACC, _H), jnp.float32),
    mesh=plsc.VectorSubcoreMesh(core_axis_name="c", subcore_axis_name="s"),
    scratch_types=[
        pltpu.VMEM((_BMAX, _EB), jnp.int32),
        pltpu.VMEM((_BMAX, _EB), jnp.int32),
        pltpu.VMEM((_EB, _H), jnp.float32),
        pltpu.VMEM((_EB, _H), jnp.float32),
        pltpu.VMEM((_ZR, _H), jnp.float32),
        pltpu.VMEM_SHARED((_ACC, _H), jnp.float32),
        pltpu.SemaphoreType.DMA,
        pltpu.SemaphoreType.DMA,
    ],
    compiler_params=pltpu.CompilerParams(use_tc_tiling_on_sc=False),
)(_seg_body)


_FR = 640


def _seg2_body(y1, p, b1v, er, out, h1a, h1b,
               src_idx, dst_idx, rows0, rows1, stage, fy, f0, f1, bvec, acc,
               sem0, sem1):
    cid = lax.axis_index("c")
    sid = lax.axis_index("s")

    def _zero(i, c):
        stage[i] = jnp.zeros((_H,), jnp.float32)
        return c
    lax.fori_loop(0, _ZR, _zero, 0)
    pltpu.sync_copy(stage, acc.at[pl.ds(sid * _ZR, _ZR)])

    def _fuse(h1_dst, nrows):
        base = sid * _FR
        pltpu.sync_copy(y1.at[pl.ds(base, nrows)], fy.at[pl.ds(0, nrows)])
        pltpu.sync_copy(p.at[0, pl.ds(base, nrows)], f0.at[pl.ds(0, nrows)])
        pltpu.sync_copy(p.at[1, pl.ds(base, nrows)], f1.at[pl.ds(0, nrows)])
        pltpu.sync_copy(b1v, bvec)
        bb = bvec[...]

        def _row(r, c):
            fy[r] = jnp.maximum(fy[r] + f0[r] + f1[r] + bb, 0.0)
            return c
        lax.fori_loop(0, nrows, _row, 0)
        pltpu.sync_copy(fy.at[pl.ds(0, nrows)], h1_dst.at[pl.ds(base, nrows)])

    @pl.when(jnp.logical_and(cid == 0, sid < _NS - 1))
    def _():
        _fuse(h1a, _FR)

    @pl.when(jnp.logical_and(cid == 0, sid == _NS - 1))
    def _():
        _fuse(h1a, _N - (_NS - 1) * _FR)

    @pl.when(jnp.logical_and(cid == 1, sid < _NS - 1))
    def _():
        _fuse(h1b, _FR)

    @pl.when(jnp.logical_and(cid == 1, sid == _NS - 1))
    def _():
        _fuse(h1b, _N - (_NS - 1) * _FR)

    plsc.subcore_barrier()

    def _edge_loop(table, base, nb):
        pltpu.sync_copy(er.at[0, pl.ds(base, nb)], src_idx.at[pl.ds(0, nb)])
        pltpu.sync_copy(er.at[1, pl.ds(base, nb)], dst_idx.at[pl.ds(0, nb)])
        pltpu.async_copy(table.at[src_idx.at[0]], rows0, sem0)

        def _step(i, c):
            j = 2 * i
            pltpu.async_copy(table.at[src_idx.at[j + 1]], rows1, sem1)
            pltpu.make_async_copy(table.at[src_idx.at[j]], rows0, sem0).wait()
            pltpu.sync_copy(rows0, acc.at[dst_idx.at[j]], add=True)

            @pl.when(j + 2 < nb)
            def _():
                pltpu.async_copy(table.at[src_idx.at[j + 2]], rows0, sem0)

            pltpu.make_async_copy(table.at[src_idx.at[j + 1]], rows1,
                                  sem1).wait()
            pltpu.sync_copy(rows1, acc.at[dst_idx.at[j + 1]], add=True)
            return c

        lax.fori_loop(0, nb // 2, _step, 0)

    @pl.when(cid == 0)
    def _():
        _edge_loop(h1a, sid * _B0, _B0)

        @pl.when(sid < _NX)
        def _():
            xb = _NS * _B0 + _NS * _B1 + sid
            pltpu.sync_copy(er.at[0, pl.ds(xb, 1)], src_idx.at[pl.ds(0, 1)])
            pltpu.sync_copy(er.at[1, pl.ds(xb, 1)], dst_idx.at[pl.ds(0, 1)])
            pltpu.async_copy(h1a.at[src_idx.at[0]], rows0, sem0).wait()
            pltpu.sync_copy(rows0, acc.at[dst_idx.at[0]], add=True)

    @pl.when(cid == 1)
    def _():
        _edge_loop(h1b, _NS * _B0 + sid * _B1, _B1)

    plsc.subcore_barrier()
    pltpu.sync_copy(acc.at[pl.ds(sid * _ZR, _ZR)],
                    out.at[cid, pl.ds(sid * _ZR, _ZR)])


_segsum2 = functools.partial(
    pl.kernel,
    out_type=(jax.ShapeDtypeStruct((_NC, _ACC, _H), jnp.float32),
              jax.ShapeDtypeStruct((_N, _H), jnp.float32),
              jax.ShapeDtypeStruct((_N, _H), jnp.float32)),
    mesh=plsc.VectorSubcoreMesh(core_axis_name="c", subcore_axis_name="s"),
    scratch_types=[
        pltpu.VMEM((_BMAX, _EB), jnp.int32),
        pltpu.VMEM((_BMAX, _EB), jnp.int32),
        pltpu.VMEM((_EB, _H), jnp.float32),
        pltpu.VMEM((_EB, _H), jnp.float32),
        pltpu.VMEM((_ZR, _H), jnp.float32),
        pltpu.VMEM((_FR, _H), jnp.float32),
        pltpu.VMEM((_FR, _H), jnp.float32),
        pltpu.VMEM((_FR, _H), jnp.float32),
        pltpu.VMEM((_H,), jnp.float32),
        pltpu.VMEM_SHARED((_ACC, _H), jnp.float32),
        pltpu.SemaphoreType.DMA,
        pltpu.SemaphoreType.DMA,
    ],
    compiler_params=pltpu.CompilerParams(use_tc_tiling_on_sc=False),
)(_seg2_body)


def _lin1_body(x_ref, w_ref, o_ref):
    o_ref[...] = jnp.dot(x_ref[...], w_ref[...],
                         preferred_element_type=jnp.float32,
                         precision=lax.Precision.HIGHEST)


def _relu_add_body(y_ref, p_ref, b_ref, o_ref):
    s = y_ref[...] + p_ref[0, :_N] + p_ref[1, :_N] + b_ref[...]
    o_ref[...] = jnp.maximum(s, 0.0)


def _out_body(h_ref, q_ref, w_ref, b_ref, o_ref):
    t = h_ref[...] + q_ref[0, :_N] + q_ref[1, :_N]
    z = jnp.dot(t, w_ref[...], preferred_element_type=jnp.float32,
                precision=lax.Precision.HIGHEST) + b_ref[...]
    z = jnp.maximum(z, 0.0)
    m = jnp.max(z, axis=1, keepdims=True)
    z = z - m
    o_ref[...] = z - jnp.log(jnp.sum(jnp.exp(z), axis=1, keepdims=True))


def kernel(x, edge_index, W1, b1, W2, b2):
    er = edge_index.reshape(2, _ER, _EB)

    y1 = pl.pallas_call(
        _lin1_body,
        out_shape=jax.ShapeDtypeStruct((_N, _H), jnp.float32),
    )(x, W1)

    p = _segsum(y1, er)

    q, h1, _h1b = _segsum2(y1, p, b1, er)

    out = pl.pallas_call(
        _out_body,
        out_shape=jax.ShapeDtypeStruct((_N, _C), jnp.float32),
    )(h1, q, W2, b2.reshape(1, _C))
    return out


# speedup vs baseline: 1.3397x; 1.0002x over previous
import functools

import jax
import jax.numpy as jnp
from jax import lax
from jax.experimental import pallas as pl
from jax.experimental.pallas import tpu as pltpu
from jax.experimental.pallas import tpu_sc as plsc

_N, _D, _H, _C, _E = 10000, 128, 16, 40, 320000
_NC, _NS = 2, 16
_EB = 128
_ER = _E // _EB
_B0 = 78
_B1 = 78
_NX = _ER - _NS * (_B0 + _B1)
_ACC = 10240
_ZR = _ACC // _NS
_BMAX = _B0 + 1


def _seg_body(table, er, out, src_idx, dst_idx, rows0, rows1, stage, acc,
              sem0, sem1):
    cid = lax.axis_index("c")
    sid = lax.axis_index("s")

    def _zero(i, c):
        stage[i] = jnp.zeros((_H,), jnp.float32)
        return c
    lax.fori_loop(0, _ZR, _zero, 0)
    pltpu.sync_copy(stage, acc.at[pl.ds(sid * _ZR, _ZR)])
    plsc.subcore_barrier()

    def _edge_loop(base, nb):
        pltpu.sync_copy(er.at[0, pl.ds(base, nb)], src_idx.at[pl.ds(0, nb)])
        pltpu.sync_copy(er.at[1, pl.ds(base, nb)], dst_idx.at[pl.ds(0, nb)])

        pltpu.async_copy(table.at[src_idx.at[0]], rows0, sem0)

        def _step(i, c):
            j = 2 * i
            pltpu.async_copy(table.at[src_idx.at[j + 1]], rows1, sem1)
            pltpu.make_async_copy(table.at[src_idx.at[j]], rows0, sem0).wait()
            pltpu.sync_copy(rows0, acc.at[dst_idx.at[j]], add=True)

            @pl.when(j + 2 < nb)
            def _():
                pltpu.async_copy(table.at[src_idx.at[j + 2]], rows0, sem0)

            pltpu.make_async_copy(table.at[src_idx.at[j + 1]], rows1,
                                  sem1).wait()
            pltpu.sync_copy(rows1, acc.at[dst_idx.at[j + 1]], add=True)
            return c

        lax.fori_loop(0, nb // 2, _step, 0)

    @pl.when(cid == 0)
    def _():
        _edge_loop(sid * _B0, _B0)

        @pl.when(sid < _NX)
        def _():
            xb = _NS * _B0 + _NS * _B1 + sid
            pltpu.sync_copy(er.at[0, pl.ds(xb, 1)], src_idx.at[pl.ds(0, 1)])
            pltpu.sync_copy(er.at[1, pl.ds(xb, 1)], dst_idx.at[pl.ds(0, 1)])
            pltpu.async_copy(table.at[src_idx.at[0]], rows0, sem0).wait()
            pltpu.sync_copy(rows0, acc.at[dst_idx.at[0]], add=True)

    @pl.when(cid == 1)
    def _():
        _edge_loop(_NS * _B0 + sid * _B1, _B1)

    plsc.subcore_barrier()

    pltpu.sync_copy(acc.at[pl.ds(sid * _ZR, _ZR)],
                    out.at[cid, pl.ds(sid * _ZR, _ZR)])


_segsum = functools.partial(
    pl.kernel,
    out_type=jax.ShapeDtypeStruct((_NC, _ACC, _H), jnp.float32),
    mesh=plsc.VectorSubcoreMesh(core_axis_name="c", subcore_axis_name="s"),
    scratch_types=[
        pltpu.VMEM((_BMAX, _EB), jnp.int32),
        pltpu.VMEM((_BMAX, _EB), jnp.int32),
        pltpu.VMEM((_EB, _H), jnp.float32),
        pltpu.VMEM((_EB, _H), jnp.float32),
        pltpu.VMEM((_ZR, _H), jnp.float32),
        pltpu.VMEM_SHARED((_ACC, _H), jnp.float32),
        pltpu.SemaphoreType.DMA,
        pltpu.SemaphoreType.DMA,
    ],
    compiler_params=pltpu.CompilerParams(use_tc_tiling_on_sc=False),
)(_seg_body)


_FR = 640


def _seg2_body(y1, p, b1v, er, out, h1a, h1b,
               src_idx, dst_idx, rows0, rows1, stage, fy, f0, f1, bvec, acc,
               sem0, sem1):
    cid = lax.axis_index("c")
    sid = lax.axis_index("s")

    def _zero(i, c):
        stage[i] = jnp.zeros((_H,), jnp.float32)
        return c
    lax.fori_loop(0, _ZR, _zero, 0)
    pltpu.sync_copy(stage, acc.at[pl.ds(sid * _ZR, _ZR)])

    def _fuse(h1_dst, nrows):
        base = sid * _FR
        pltpu.sync_copy(y1.at[pl.ds(base, nrows)], fy.at[pl.ds(0, nrows)])
        pltpu.sync_copy(p.at[0, pl.ds(base, nrows)], f0.at[pl.ds(0, nrows)])
        pltpu.sync_copy(p.at[1, pl.ds(base, nrows)], f1.at[pl.ds(0, nrows)])
        pltpu.sync_copy(b1v, bvec)
        bb = bvec[...]

        def _row(r, c):
            fy[r] = jnp.maximum(fy[r] + f0[r] + f1[r] + bb, 0.0)
            return c
        lax.fori_loop(0, nrows, _row, 0)
        pltpu.sync_copy(fy.at[pl.ds(0, nrows)], h1_dst.at[pl.ds(base, nrows)])

    @pl.when(jnp.logical_and(cid == 0, sid < _NS - 1))
    def _():
        _fuse(h1a, _FR)

    @pl.when(jnp.logical_and(cid == 0, sid == _NS - 1))
    def _():
        _fuse(h1a, _N - (_NS - 1) * _FR)

    @pl.when(jnp.logical_and(cid == 1, sid < _NS - 1))
    def _():
        _fuse(h1b, _FR)

    @pl.when(jnp.logical_and(cid == 1, sid == _NS - 1))
    def _():
        _fuse(h1b, _N - (_NS - 1) * _FR)

    plsc.subcore_barrier()

    def _edge_loop(table, base, nb):
        pltpu.sync_copy(er.at[0, pl.ds(base, nb)], src_idx.at[pl.ds(0, nb)])
        pltpu.sync_copy(er.at[1, pl.ds(base, nb)], dst_idx.at[pl.ds(0, nb)])
        pltpu.async_copy(table.at[src_idx.at[0]], rows0, sem0)

        def _step(i, c):
            j = 2 * i
            pltpu.async_copy(table.at[src_idx.at[j + 1]], rows1, sem1)
            pltpu.make_async_copy(table.at[src_idx.at[j]], rows0, sem0).wait()
            pltpu.sync_copy(rows0, acc.at[dst_idx.at[j]], add=True)

            @pl.when(j + 2 < nb)
            def _():
                pltpu.async_copy(table.at[src_idx.at[j + 2]], rows0, sem0)

            pltpu.make_async_copy(table.at[src_idx.at[j + 1]], rows1,
                                  sem1).wait()
            pltpu.sync_copy(rows1, acc.at[dst_idx.at[j + 1]], add=True)
            return c

        lax.fori_loop(0, nb // 2, _step, 0)

    @pl.when(cid == 0)
    def _():
        _edge_loop(h1a, sid * _B0, _B0)

        @pl.when(sid < _NX)
        def _():
            xb = _NS * _B0 + _NS * _B1 + sid
            pltpu.sync_copy(er.at[0, pl.ds(xb, 1)], src_idx.at[pl.ds(0, 1)])
            pltpu.sync_copy(er.at[1, pl.ds(xb, 1)], dst_idx.at[pl.ds(0, 1)])
            pltpu.async_copy(h1a.at[src_idx.at[0]], rows0, sem0).wait()
            pltpu.sync_copy(rows0, acc.at[dst_idx.at[0]], add=True)

    @pl.when(cid == 1)
    def _():
        _edge_loop(h1b, _NS * _B0 + sid * _B1, _B1)

    plsc.subcore_barrier()
    pltpu.sync_copy(acc.at[pl.ds(sid * _ZR, _ZR)],
                    out.at[cid, pl.ds(sid * _ZR, _ZR)])


_segsum2 = functools.partial(
    pl.kernel,
    out_type=(jax.ShapeDtypeStruct((_NC, _ACC, _H), jnp.float32),
              jax.ShapeDtypeStruct((_N, _H), jnp.float32),
              jax.ShapeDtypeStruct((_N, _H), jnp.float32)),
    mesh=plsc.VectorSubcoreMesh(core_axis_name="c", subcore_axis_name="s"),
    scratch_types=[
        pltpu.VMEM((_BMAX, _EB), jnp.int32),
        pltpu.VMEM((_BMAX, _EB), jnp.int32),
        pltpu.VMEM((_EB, _H), jnp.float32),
        pltpu.VMEM((_EB, _H), jnp.float32),
        pltpu.VMEM((_ZR, _H), jnp.float32),
        pltpu.VMEM((_FR, _H), jnp.float32),
        pltpu.VMEM((_FR, _H), jnp.float32),
        pltpu.VMEM((_FR, _H), jnp.float32),
        pltpu.VMEM((_H,), jnp.float32),
        pltpu.VMEM_SHARED((_ACC, _H), jnp.float32),
        pltpu.SemaphoreType.DMA,
        pltpu.SemaphoreType.DMA,
    ],
    compiler_params=pltpu.CompilerParams(use_tc_tiling_on_sc=False),
)(_seg2_body)


def _lin1_body(x_ref, w_ref, o_ref):
    o_ref[...] = jnp.dot(x_ref[...], w_ref[...],
                         preferred_element_type=jnp.float32,
                         precision=lax.Precision.HIGHEST)


def _out_body(h_ref, q_ref, w_ref, b_ref, o_ref):
    t = h_ref[...] + q_ref[0] + q_ref[1]
    z = jnp.dot(t, w_ref[...], preferred_element_type=jnp.float32,
                precision=lax.Precision.HIGHEST) + b_ref[...]
    z = jnp.maximum(z, 0.0)
    m = jnp.max(z, axis=1, keepdims=True)
    z = z - m
    o_ref[...] = z - jnp.log(jnp.sum(jnp.exp(z), axis=1, keepdims=True))


def kernel(x, edge_index, W1, b1, W2, b2):
    er = edge_index.reshape(2, _ER, _EB)

    y1 = pl.pallas_call(
        _lin1_body,
        out_shape=jax.ShapeDtypeStruct((_N, _H), jnp.float32),
    )(x, W1)

    p = _segsum(y1, er)

    q, h1, _h1b = _segsum2(y1, p, b1, er)

    blk = 1000
    out = pl.pallas_call(
        _out_body,
        grid=(_N // blk,),
        in_specs=[
            pl.BlockSpec((blk, _H), lambda i: (i, 0)),
            pl.BlockSpec((2, blk, _H), lambda i: (0, i, 0)),
            pl.BlockSpec((_H, _C), lambda i: (0, 0)),
            pl.BlockSpec((1, _C), lambda i: (0, 0)),
        ],
        out_specs=pl.BlockSpec((blk, _C), lambda i: (i, 0)),
        out_shape=jax.ShapeDtypeStruct((_N, _C), jnp.float32),
    )(h1, q, W2, b2.reshape(1, _C))
    return out
